# baseline (speedup 1.0000x reference)
import functools

import jax
import jax.numpy as jnp
from jax import lax
from jax.experimental import pallas as pl
from jax.experimental.pallas import tpu as pltpu
from jax.experimental.pallas import tpu_sc as plsc

_NW = 32
_CH = 128


def _gather_call(n_chunks, feat):
    epw = n_chunks * _CH
    bp = _NW * epw
    nb = 2
    mesh = plsc.VectorSubcoreMesh(core_axis_name="c", subcore_axis_name="s")

    @functools.partial(
        pl.kernel,
        mesh=mesh,
        out_type=jax.ShapeDtypeStruct((bp, feat), jnp.float32),
        scratch_types=(
            [pltpu.VMEM((n_chunks, _CH), jnp.int32)]
            + [pltpu.VMEM((_CH, feat), jnp.float32)] * nb
            + [pltpu.SemaphoreType.DMA] * (2 * nb)
        ),
        compiler_params=pltpu.CompilerParams(use_tc_tiling_on_sc=False),
    )
    def gk(table_hbm, idx_hbm, out_hbm, idx_v, *rest):
        bufs = rest[:nb]
        gs = rest[nb:2 * nb]
        ws = rest[2 * nb:]
        wid = lax.axis_index("s") * 2 + lax.axis_index("c")
        pltpu.sync_copy(idx_hbm.at[wid], idx_v)
        base = wid * epw

        def g(j, s):
            return pltpu.make_async_copy(
                table_hbm.at[idx_v.at[j]], bufs[s], gs[s])

        def w(j, s):
            return pltpu.make_async_copy(
                bufs[s], out_hbm.at[pl.ds(base + j * _CH, _CH)], ws[s])

        g(0, 0).start()
        g(1, 1).start()

        def body(p, carry):
            for c in range(2):
                j = 2 * p + c

                @pl.when(j >= 2)
                def _():
                    w(j - 2, c).wait()

                @pl.when(j + 2 < n_chunks)
                def _():
                    g(j + 2, c).start()

                g(j, c).wait()
                w(j, c).start()
            return carry

        lax.fori_loop(0, n_chunks // 2, body, 0)
        for c in range(2):
            w(n_chunks - 2 + c, c).wait()

    return gk


def _embed(atom_fea, w, b, ba):
    n, orig = atom_fea.shape
    a = w.shape[1]
    nblk = n // ba

    def body(x_ref, w_ref, b_ref, o_ref):
        o_ref[...] = (
            jnp.dot(x_ref[...], w_ref[...], preferred_element_type=jnp.float32)
            + b_ref[...]
        )

    return pl.pallas_call(
        body,
        grid=(nblk,),
        in_specs=[
            pl.BlockSpec((ba, orig), lambda i: (i, 0)),
            pl.BlockSpec((orig, a), lambda i: (0, 0)),
            pl.BlockSpec((1, a), lambda i: (0, 0)),
        ],
        out_specs=pl.BlockSpec((ba, a), lambda i: (i, 0)),
        out_shape=jax.ShapeDtypeStruct((n, a), jnp.float32),
    )(atom_fea, w, b[None, :])


def _dgt(lhs, rhs):
    return lax.dot_general(
        lhs, rhs, (((0,), (0,)), ((), ())), preferred_element_type=jnp.float32
    )


def _stats(x, xgp, fp, m, ba):
    n, a = x.shape
    nbr = fp.shape[1] // 2
    pb = ba * m // 2
    nblk = n // ba
    g = m // 2

    def body(x_ref, xg_ref, f_ref, ss, sn, nn, nf, sf, ff, us, un, uf):
        @pl.when(pl.program_id(0) == 0)
        def _():
            for r in (ss, sn, nn, nf, sf, ff, us, un, uf):
                r[...] = jnp.zeros_like(r)

        xb = x_ref[...]
        xgb = xg_ref[...]
        fb = f_ref[...]
        x2b = jnp.concatenate([xb, xb], axis=1)
        xrep = jnp.broadcast_to(
            x2b[:, None, :], (ba, g, 2 * a)).reshape(pb, 2 * a)
        pnn = _dgt(xgb, xgb)
        pnf = _dgt(xgb, fb)
        pff = _dgt(fb, fb)
        psn = _dgt(xrep, xgb)
        psf = _dgt(xrep, fb)
        ss[...] += float(m) * _dgt(xb, xb)
        sn[...] += psn[:a, :a] + psn[a:, a:]
        nn[...] += pnn[:a, :a] + pnn[a:, a:]
        nf[...] += pnf[:a, :nbr] + pnf[a:, nbr:]
        sf[...] += psf[:a, :nbr] + psf[a:, nbr:]
        ff[...] += pff[:nbr, :nbr] + pff[nbr:, nbr:]
        us[...] += jnp.broadcast_to(
            float(m) * jnp.sum(xb, axis=0, keepdims=True), (8, a))
        unw = jnp.sum(xgb, axis=0, keepdims=True)
        un[...] += jnp.broadcast_to(unw[:, :a] + unw[:, a:], (8, a))
        ufw = jnp.sum(fb, axis=0, keepdims=True)
        uf[...] += jnp.broadcast_to(ufw[:, :nbr] + ufw[:, nbr:], (8, nbr))

    acc = lambda i: (0, 0)
    outs = pl.pallas_call(
        body,
        grid=(nblk,),
        in_specs=[
            pl.BlockSpec((ba, a), lambda i: (i, 0)),
            pl.BlockSpec((pb, 2 * a), lambda i: (i, 0)),
            pl.BlockSpec((pb, 2 * nbr), lambda i: (i, 0)),
        ],
        out_specs=[
            pl.BlockSpec((a, a), acc),
            pl.BlockSpec((a, a), acc),
            pl.BlockSpec((a, a), acc),
            pl.BlockSpec((a, nbr), acc),
            pl.BlockSpec((a, nbr), acc),
            pl.BlockSpec((nbr, nbr), acc),
            pl.BlockSpec((8, a), acc),
            pl.BlockSpec((8, a), acc),
            pl.BlockSpec((8, nbr), acc),
        ],
        out_shape=[
            jax.ShapeDtypeStruct((a, a), jnp.float32),
            jax.ShapeDtypeStruct((a, a), jnp.float32),
            jax.ShapeDtypeStruct((a, a), jnp.float32),
            jax.ShapeDtypeStruct((a, nbr), jnp.float32),
            jax.ShapeDtypeStruct((a, nbr), jnp.float32),
            jax.ShapeDtypeStruct((nbr, nbr), jnp.float32),
            jax.ShapeDtypeStruct((8, a), jnp.float32),
            jax.ShapeDtypeStruct((8, a), jnp.float32),
            jax.ShapeDtypeStruct((8, nbr), jnp.float32),
        ],
    )(x, xgp, fp)
    return outs


def _main(x, xgp, fp, wsf, wsc, wn2f, wn2c, wf2f, wf2c, bf, bc, m, ba):
    n, a = x.shape
    nbr = fp.shape[1] // 2
    pb = ba * m // 2
    nblk = n // ba
    g = m // 2

    def body(x_ref, xg_ref, f_ref, wsf_r, wsc_r, wnf_r, wnc_r, wff_r, wfc_r,
             bf_r, bc_r, y_ref, s_ref, q_ref):
        xb = x_ref[...]
        xgb = xg_ref[...]
        fb = f_ref[...]
        dot = lambda p, q: jnp.dot(p, q, preferred_element_type=jnp.float32)
        pf2 = dot(xb, wsf_r[...]) + bf_r[...]
        pc2 = dot(xb, wsc_r[...]) + bc_r[...]
        gf = (dot(xgb, wnf_r[...]) + dot(fb, wff_r[...])).reshape(ba, g, 2 * a) \
            + pf2[:, None, :]
        gc = (dot(xgb, wnc_r[...]) + dot(fb, wfc_r[...])).reshape(ba, g, 2 * a) \
            + pc2[:, None, :]
        prod = jax.nn.sigmoid(gf) * jnp.where(gc >= 0.0, gc, 0.01 * gc)
        z = jnp.sum(prod, axis=1)
        y = z[:, :a] + z[:, a:]
        y_ref[...] = y

        @pl.when(pl.program_id(0) == 0)
        def _():
            s_ref[...] = jnp.zeros_like(s_ref)
            q_ref[...] = jnp.zeros_like(q_ref)

        s_ref[...] += jnp.broadcast_to(jnp.sum(y, axis=0, keepdims=True), (8, a))
        q_ref[...] += jnp.broadcast_to(
            jnp.sum(y * y, axis=0, keepdims=True), (8, a))

    full = lambda i: (0, 0)
    return pl.pallas_call(
        body,
        grid=(nblk,),
        in_specs=[
            pl.BlockSpec((ba, a), lambda i: (i, 0)),
            pl.BlockSpec((pb, 2 * a), lambda i: (i, 0)),
            pl.BlockSpec((pb, 2 * nbr), lambda i: (i, 0)),
            pl.BlockSpec((a, 2 * a), full),
            pl.BlockSpec((a, 2 * a), full),
            pl.BlockSpec((2 * a, 2 * a), full),
            pl.BlockSpec((2 * a, 2 * a), full),
            pl.BlockSpec((2 * nbr, 2 * a), full),
            pl.BlockSpec((2 * nbr, 2 * a), full),
            pl.BlockSpec((1, 2 * a), full),
            pl.BlockSpec((1, 2 * a), full),
        ],
        out_specs=[
            pl.BlockSpec((ba, a), lambda i: (i, 0)),
            pl.BlockSpec((8, a), full),
            pl.BlockSpec((8, a), full),
        ],
        out_shape=[
            jax.ShapeDtypeStruct((n, a), jnp.float32),
            jax.ShapeDtypeStruct((8, a), jnp.float32),
            jax.ShapeDtypeStruct((8, a), jnp.float32),
        ],
    )(x, xgp, fp, wsf, wsc, wn2f, wn2c, wf2f, wf2c, bf, bc)


def _elem(x, y, s2, t2, ba):
    n, a = x.shape
    nblk = n // ba

    def body(x_ref, y_ref, s_ref, t_ref, o_ref):
        t = x_ref[...] + y_ref[...] * s_ref[...] + t_ref[...]
        o_ref[...] = jnp.where(t >= 0.0, t, 0.01 * t)

    return pl.pallas_call(
        body,
        grid=(nblk,),
        in_specs=[
            pl.BlockSpec((ba, a), lambda i: (i, 0)),
            pl.BlockSpec((ba, a), lambda i: (i, 0)),
            pl.BlockSpec((1, a), lambda i: (0, 0)),
            pl.BlockSpec((1, a), lambda i: (0, 0)),
        ],
        out_specs=pl.BlockSpec((ba, a), lambda i: (i, 0)),
        out_shape=jax.ShapeDtypeStruct((n, a), jnp.float32),
    )(x, y, s2, t2)


def kernel(atom_fea, nbr_fea, nbr_fea_idx, nbr_fea_offset, crystal_atom_idx,
           atom_pos, nbr_pos, atom_pos_idx, cells, fixed_atom_mask,
           atom_pos_final, W_emb, b_emb, W_full, b_full, bn1_g, bn1_b,
           bn2_g, bn2_b):
    n, _ = atom_fea.shape
    m = nbr_fea_idx.shape[1]
    nbr = nbr_fea.shape[2]
    a = W_emb.shape[1]
    nconv = W_full.shape[0]
    e = n * m
    r1 = float(e)
    eps = 1e-5

    if n % 1000 == 0:
        ba = 1000
    elif n % 400 == 0:
        ba = 400
    else:
        ba = 8
    fp = nbr_fea.reshape(e // 2, 2 * nbr)
    idx = nbr_fea_idx.astype(jnp.int32).reshape(-1)
    per = _NW * _CH
    n_chunks = ((-(-e // per)) + 7) // 8 * 8
    bp = n_chunks * per
    idx2 = jnp.pad(idx, (0, bp - e)).reshape(_NW, n_chunks, _CH)
    gather = _gather_call(n_chunks, a)

    def diag2(wsub):
        k = wsub.shape[0]
        z = jnp.zeros((2 * k, 2 * a), jnp.float32)
        return z.at[:k, :a].set(wsub).at[k:, a:].set(wsub)

    x = _embed(atom_fea, W_emb, b_emb, 2000 if n % 2000 == 0 else ba)

    for i in range(nconv):
        xgp = gather(x, idx2).reshape(bp // 2, 2 * a)
        ss, sn, nn, nf, sf, ff, us, un, uf = _stats(x, xgp, fp, m, ba)

        top = jnp.concatenate([ss, sn, sf], axis=1)
        mid = jnp.concatenate([sn.T, nn, nf], axis=1)
        bot = jnp.concatenate([sf.T, nf.T, ff], axis=1)
        s_mat = jnp.concatenate([top, mid, bot], axis=0)
        sums = jnp.concatenate([us[0], un[0], uf[0]])
        w = W_full[i]
        b = b_full[i]
        u = (sums @ w) / r1
        q = jnp.sum(w * (s_mat @ w), axis=0)
        var1 = q / r1 - u * u
        s1 = bn1_g[i] / jnp.sqrt(var1 + eps)
        t1 = bn1_b[i] - (u + b) * s1
        wp = w * s1[None, :]
        bpv = (b * s1 + t1)[None, :]

        dup = lambda v: jnp.concatenate([v, v], axis=1)
        y, ysum, ysq = _main(
            x, xgp, fp,
            dup(wp[:a, :a]), dup(wp[:a, a:]),
            diag2(wp[a:2 * a, :a]), diag2(wp[a:2 * a, a:]),
            diag2(wp[2 * a:, :a]), diag2(wp[2 * a:, a:]),
            dup(bpv[:, :a]), dup(bpv[:, a:]), m, ba)

        mean2 = ysum[0] / float(n)
        var2 = ysq[0] / float(n) - mean2 * mean2
        s2 = bn2_g[i] / jnp.sqrt(var2 + eps)
        t2 = bn2_b[i] - mean2 * s2
        x = _elem(x, y, s2[None, :], t2[None, :], ba)

    return x


# speedup vs baseline: 1.4322x; 1.4322x over previous
import functools

import jax
import jax.numpy as jnp
from jax import lax
from jax.experimental import pallas as pl
from jax.experimental.pallas import tpu as pltpu
from jax.experimental.pallas import tpu_sc as plsc

_NW = 32
_CH = 128


def _gather_call(n_chunks, feat):
    epw = n_chunks * _CH
    bp = _NW * epw
    nb = 2
    mesh = plsc.VectorSubcoreMesh(core_axis_name="c", subcore_axis_name="s")

    @functools.partial(
        pl.kernel,
        mesh=mesh,
        out_type=jax.ShapeDtypeStruct((bp, feat), jnp.float32),
        scratch_types=(
            [pltpu.VMEM((n_chunks, _CH), jnp.int32)]
            + [pltpu.VMEM((_CH, feat), jnp.float32)] * nb
            + [pltpu.SemaphoreType.DMA] * (2 * nb)
        ),
        compiler_params=pltpu.CompilerParams(use_tc_tiling_on_sc=False),
    )
    def gk(table_hbm, idx_hbm, out_hbm, idx_v, *rest):
        bufs = rest[:nb]
        gs = rest[nb:2 * nb]
        ws = rest[2 * nb:]
        wid = lax.axis_index("s") * 2 + lax.axis_index("c")
        pltpu.sync_copy(idx_hbm.at[wid], idx_v)
        base = wid * epw

        def g(j, s):
            return pltpu.make_async_copy(
                table_hbm.at[idx_v.at[j]], bufs[s], gs[s])

        def w(j, s):
            return pltpu.make_async_copy(
                bufs[s], out_hbm.at[pl.ds(base + j * _CH, _CH)], ws[s])

        g(0, 0).start()

        def body(p, carry):
            j0 = 2 * p
            g(j0 + 1, 1).start()
            g(j0, 0).wait()
            pltpu.sync_copy(bufs[0], out_hbm.at[pl.ds(base + j0 * _CH, _CH)])

            @pl.when(p + 1 < n_chunks // 2)
            def _():
                g(j0 + 2, 0).start()

            g(j0 + 1, 1).wait()
            pltpu.sync_copy(
                bufs[1], out_hbm.at[pl.ds(base + (j0 + 1) * _CH, _CH)])
            return carry

        lax.fori_loop(0, n_chunks // 2, body, 0)

    return gk


def _embed(atom_fea, w, b, ba):
    n, orig = atom_fea.shape
    a = w.shape[1]
    nblk = n // ba

    def body(x_ref, w_ref, b_ref, o_ref):
        o_ref[...] = (
            jnp.dot(x_ref[...], w_ref[...], preferred_element_type=jnp.float32)
            + b_ref[...]
        )

    return pl.pallas_call(
        body,
        grid=(nblk,),
        in_specs=[
            pl.BlockSpec((ba, orig), lambda i: (i, 0)),
            pl.BlockSpec((orig, a), lambda i: (0, 0)),
            pl.BlockSpec((1, a), lambda i: (0, 0)),
        ],
        out_specs=pl.BlockSpec((ba, a), lambda i: (i, 0)),
        out_shape=jax.ShapeDtypeStruct((n, a), jnp.float32),
    )(atom_fea, w, b[None, :])


def _dgt(lhs, rhs):
    return lax.dot_general(
        lhs, rhs, (((0,), (0,)), ((), ())), preferred_element_type=jnp.float32
    )


def _stats(x, xgp, fp, m, ba):
    n, a = x.shape
    nbr = fp.shape[1] // 2
    pb = ba * m // 2
    nblk = n // ba
    g = m // 2

    def body(x_ref, xg_ref, f_ref, ss, sn, nn, nf, sf, ff, us, un, uf):
        @pl.when(pl.program_id(0) == 0)
        def _():
            for r in (ss, sn, nn, nf, sf, ff, us, un, uf):
                r[...] = jnp.zeros_like(r)

        xb = x_ref[...]
        xgb = xg_ref[...]
        fb = f_ref[...]
        x2b = jnp.concatenate([xb, xb], axis=1)
        xrep = jnp.broadcast_to(
            x2b[:, None, :], (ba, g, 2 * a)).reshape(pb, 2 * a)
        pnn = _dgt(xgb, xgb)
        pnf = _dgt(xgb, fb)
        pff = _dgt(fb, fb)
        psn = _dgt(xrep, xgb)
        psf = _dgt(xrep, fb)
        ss[...] += float(m) * _dgt(xb, xb)
        sn[...] += psn[:a, :a] + psn[a:, a:]
        nn[...] += pnn[:a, :a] + pnn[a:, a:]
        nf[...] += pnf[:a, :nbr] + pnf[a:, nbr:]
        sf[...] += psf[:a, :nbr] + psf[a:, nbr:]
        ff[...] += pff[:nbr, :nbr] + pff[nbr:, nbr:]
        us[...] += jnp.broadcast_to(
            float(m) * jnp.sum(xb, axis=0, keepdims=True), (8, a))
        unw = jnp.sum(xgb, axis=0, keepdims=True)
        un[...] += jnp.broadcast_to(unw[:, :a] + unw[:, a:], (8, a))
        ufw = jnp.sum(fb, axis=0, keepdims=True)
        uf[...] += jnp.broadcast_to(ufw[:, :nbr] + ufw[:, nbr:], (8, nbr))

    acc = lambda i: (0, 0)
    outs = pl.pallas_call(
        body,
        grid=(nblk,),
        in_specs=[
            pl.BlockSpec((ba, a), lambda i: (i, 0)),
            pl.BlockSpec((pb, 2 * a), lambda i: (i, 0)),
            pl.BlockSpec((pb, 2 * nbr), lambda i: (i, 0)),
        ],
        out_specs=[
            pl.BlockSpec((a, a), acc),
            pl.BlockSpec((a, a), acc),
            pl.BlockSpec((a, a), acc),
            pl.BlockSpec((a, nbr), acc),
            pl.BlockSpec((a, nbr), acc),
            pl.BlockSpec((nbr, nbr), acc),
            pl.BlockSpec((8, a), acc),
            pl.BlockSpec((8, a), acc),
            pl.BlockSpec((8, nbr), acc),
        ],
        out_shape=[
            jax.ShapeDtypeStruct((a, a), jnp.float32),
            jax.ShapeDtypeStruct((a, a), jnp.float32),
            jax.ShapeDtypeStruct((a, a), jnp.float32),
            jax.ShapeDtypeStruct((a, nbr), jnp.float32),
            jax.ShapeDtypeStruct((a, nbr), jnp.float32),
            jax.ShapeDtypeStruct((nbr, nbr), jnp.float32),
            jax.ShapeDtypeStruct((8, a), jnp.float32),
            jax.ShapeDtypeStruct((8, a), jnp.float32),
            jax.ShapeDtypeStruct((8, nbr), jnp.float32),
        ],
    )(x, xgp, fp)
    return outs


def _main(x, xgp, fp, wsf, wsc, wn2f, wn2c, wf2f, wf2c, bf, bc, m, ba):
    n, a = x.shape
    nbr = fp.shape[1] // 2
    pb = ba * m // 2
    nblk = n // ba
    g = m // 2

    def body(x_ref, xg_ref, f_ref, wsf_r, wsc_r, wnf_r, wnc_r, wff_r, wfc_r,
             bf_r, bc_r, y_ref, s_ref, q_ref):
        xb = x_ref[...]
        xgb = xg_ref[...]
        fb = f_ref[...]
        dot = lambda p, q: jnp.dot(p, q, preferred_element_type=jnp.float32)
        pf2 = dot(xb, wsf_r[...]) + bf_r[...]
        pc2 = dot(xb, wsc_r[...]) + bc_r[...]
        gf = (dot(xgb, wnf_r[...]) + dot(fb, wff_r[...])).reshape(ba, g, 2 * a) \
            + pf2[:, None, :]
        gc = (dot(xgb, wnc_r[...]) + dot(fb, wfc_r[...])).reshape(ba, g, 2 * a) \
            + pc2[:, None, :]
        prod = jax.nn.sigmoid(gf) * jnp.where(gc >= 0.0, gc, 0.01 * gc)
        z = jnp.sum(prod, axis=1)
        y = z[:, :a] + z[:, a:]
        y_ref[...] = y

        @pl.when(pl.program_id(0) == 0)
        def _():
            s_ref[...] = jnp.zeros_like(s_ref)
            q_ref[...] = jnp.zeros_like(q_ref)

        s_ref[...] += jnp.broadcast_to(jnp.sum(y, axis=0, keepdims=True), (8, a))
        q_ref[...] += jnp.broadcast_to(
            jnp.sum(y * y, axis=0, keepdims=True), (8, a))

    full = lambda i: (0, 0)
    return pl.pallas_call(
        body,
        grid=(nblk,),
        in_specs=[
            pl.BlockSpec((ba, a), lambda i: (i, 0)),
            pl.BlockSpec((pb, 2 * a), lambda i: (i, 0)),
            pl.BlockSpec((pb, 2 * nbr), lambda i: (i, 0)),
            pl.BlockSpec((a, 2 * a), full),
            pl.BlockSpec((a, 2 * a), full),
            pl.BlockSpec((2 * a, 2 * a), full),
            pl.BlockSpec((2 * a, 2 * a), full),
            pl.BlockSpec((2 * nbr, 2 * a), full),
            pl.BlockSpec((2 * nbr, 2 * a), full),
            pl.BlockSpec((1, 2 * a), full),
            pl.BlockSpec((1, 2 * a), full),
        ],
        out_specs=[
            pl.BlockSpec((ba, a), lambda i: (i, 0)),
            pl.BlockSpec((8, a), full),
            pl.BlockSpec((8, a), full),
        ],
        out_shape=[
            jax.ShapeDtypeStruct((n, a), jnp.float32),
            jax.ShapeDtypeStruct((8, a), jnp.float32),
            jax.ShapeDtypeStruct((8, a), jnp.float32),
        ],
    )(x, xgp, fp, wsf, wsc, wn2f, wn2c, wf2f, wf2c, bf, bc)


def _elem(x, y, s2, t2, ba):
    n, a = x.shape
    nblk = n // ba

    def body(x_ref, y_ref, s_ref, t_ref, o_ref):
        t = x_ref[...] + y_ref[...] * s_ref[...] + t_ref[...]
        o_ref[...] = jnp.where(t >= 0.0, t, 0.01 * t)

    return pl.pallas_call(
        body,
        grid=(nblk,),
        in_specs=[
            pl.BlockSpec((ba, a), lambda i: (i, 0)),
            pl.BlockSpec((ba, a), lambda i: (i, 0)),
            pl.BlockSpec((1, a), lambda i: (0, 0)),
            pl.BlockSpec((1, a), lambda i: (0, 0)),
        ],
        out_specs=pl.BlockSpec((ba, a), lambda i: (i, 0)),
        out_shape=jax.ShapeDtypeStruct((n, a), jnp.float32),
    )(x, y, s2, t2)


def kernel(atom_fea, nbr_fea, nbr_fea_idx, nbr_fea_offset, crystal_atom_idx,
           atom_pos, nbr_pos, atom_pos_idx, cells, fixed_atom_mask,
           atom_pos_final, W_emb, b_emb, W_full, b_full, bn1_g, bn1_b,
           bn2_g, bn2_b):
    n, _ = atom_fea.shape
    m = nbr_fea_idx.shape[1]
    nbr = nbr_fea.shape[2]
    a = W_emb.shape[1]
    nconv = W_full.shape[0]
    e = n * m
    r1 = float(e)
    eps = 1e-5

    if n % 1000 == 0:
        ba = 1000
    elif n % 400 == 0:
        ba = 400
    else:
        ba = 8
    fp = nbr_fea.reshape(e // 2, 2 * nbr)
    idx = nbr_fea_idx.astype(jnp.int32).reshape(-1)
    per = _NW * _CH
    n_chunks = ((-(-e // per)) + 1) // 2 * 2
    bp = n_chunks * per
    idx2 = jnp.pad(idx, (0, bp - e)).reshape(_NW, n_chunks, _CH)
    gather = _gather_call(n_chunks, a)

    def diag2(wsub):
        k = wsub.shape[0]
        z = jnp.zeros((2 * k, 2 * a), jnp.float32)
        return z.at[:k, :a].set(wsub).at[k:, a:].set(wsub)

    x = _embed(atom_fea, W_emb, b_emb, 2000 if n % 2000 == 0 else ba)

    for i in range(nconv):
        xgp = gather(x, idx2).reshape(bp // 2, 2 * a)
        ss, sn, nn, nf, sf, ff, us, un, uf = _stats(x, xgp, fp, m, ba)

        top = jnp.concatenate([ss, sn, sf], axis=1)
        mid = jnp.concatenate([sn.T, nn, nf], axis=1)
        bot = jnp.concatenate([sf.T, nf.T, ff], axis=1)
        s_mat = jnp.concatenate([top, mid, bot], axis=0)
        sums = jnp.concatenate([us[0], un[0], uf[0]])
        w = W_full[i]
        b = b_full[i]
        u = (sums @ w) / r1
        q = jnp.sum(w * (s_mat @ w), axis=0)
        var1 = q / r1 - u * u
        s1 = bn1_g[i] / jnp.sqrt(var1 + eps)
        t1 = bn1_b[i] - (u + b) * s1
        wp = w * s1[None, :]
        bpv = (b * s1 + t1)[None, :]

        dup = lambda v: jnp.concatenate([v, v], axis=1)
        y, ysum, ysq = _main(
            x, xgp, fp,
            dup(wp[:a, :a]), dup(wp[:a, a:]),
            diag2(wp[a:2 * a, :a]), diag2(wp[a:2 * a, a:]),
            diag2(wp[2 * a:, :a]), diag2(wp[2 * a:, a:]),
            dup(bpv[:, :a]), dup(bpv[:, a:]), m, ba)

        mean2 = ysum[0] / float(n)
        var2 = ysq[0] / float(n) - mean2 * mean2
        s2 = bn2_g[i] / jnp.sqrt(var2 + eps)
        t2 = bn2_b[i] - mean2 * s2
        x = _elem(x, y, s2[None, :], t2[None, :], ba)

    return x
